# baseline (speedup 1.0000x reference)
import functools

import jax
import jax.numpy as jnp
from jax import lax
from jax.experimental import pallas as pl
from jax.experimental.pallas import tpu as pltpu
from jax.experimental.pallas import tpu_sc as plsc

DIM = 64
L = 16
SUB = 128


def _sc_partial_scores(ent_re, ent_im, rel_re, rel_im, h_idx, r_idx, t_idx):
    b_total = h_idx.shape[0]
    info = plsc.get_sparse_core_info()
    nw = info.num_cores * info.num_subcores
    chunk = b_total // nw
    nsub = chunk // SUB
    assert chunk % SUB == 0

    mesh = plsc.VectorSubcoreMesh(core_axis_name="c", subcore_axis_name="s")

    @functools.partial(
        pl.kernel,
        mesh=mesh,
        compiler_params=pltpu.CompilerParams(use_tc_tiling_on_sc=False),
        out_type=jax.ShapeDtypeStruct((b_total, L), jnp.float32),
        scratch_types=[
            pltpu.VMEM((chunk,), jnp.int32),
            pltpu.VMEM((chunk,), jnp.int32),
            pltpu.VMEM((chunk,), jnp.int32),
            pltpu.VMEM((SUB, DIM), jnp.float32),
            pltpu.VMEM((SUB, DIM), jnp.float32),
            pltpu.VMEM((SUB, DIM), jnp.float32),
            pltpu.VMEM((SUB, DIM), jnp.float32),
            pltpu.VMEM((SUB, DIM), jnp.float32),
            pltpu.VMEM((SUB, DIM), jnp.float32),
            pltpu.VMEM((SUB, L), jnp.float32),
            pltpu.SemaphoreType.DMA,
        ],
    )
    def sc_kernel(ent_re_hbm, ent_im_hbm, rel_re_hbm, rel_im_hbm,
                  h_hbm, r_hbm, t_hbm, out_hbm,
                  h_v, r_v, t_v, hre_v, him_v, tre_v, tim_v, rre_v, rim_v,
                  part_v, sem):
        wid = lax.axis_index("s") * info.num_cores + lax.axis_index("c")
        base = wid * chunk
        pltpu.sync_copy(h_hbm.at[pl.ds(base, chunk)], h_v)
        pltpu.sync_copy(r_hbm.at[pl.ds(base, chunk)], r_v)
        pltpu.sync_copy(t_hbm.at[pl.ds(base, chunk)], t_v)

        def sub_body(s, carry):
            off = s * SUB
            cps = [
                pltpu.async_copy(ent_re_hbm.at[h_v.at[pl.ds(off, SUB)]], hre_v, sem),
                pltpu.async_copy(ent_im_hbm.at[h_v.at[pl.ds(off, SUB)]], him_v, sem),
                pltpu.async_copy(ent_re_hbm.at[t_v.at[pl.ds(off, SUB)]], tre_v, sem),
                pltpu.async_copy(ent_im_hbm.at[t_v.at[pl.ds(off, SUB)]], tim_v, sem),
                pltpu.async_copy(rel_re_hbm.at[r_v.at[pl.ds(off, SUB)]], rre_v, sem),
                pltpu.async_copy(rel_im_hbm.at[r_v.at[pl.ds(off, SUB)]], rim_v, sem),
            ]
            for cp in cps:
                cp.wait()

            def row_body(i, c2):
                acc = jnp.zeros((L,), jnp.float32)
                for c in range(DIM // L):
                    sl = pl.ds(c * L, L)
                    hre = hre_v[i, sl]
                    him = him_v[i, sl]
                    tre = tre_v[i, sl]
                    tim = tim_v[i, sl]
                    rre = rre_v[i, sl]
                    rim = rim_v[i, sl]
                    acc = acc + rre * (hre * tre + him * tim) + rim * (hre * tim - him * tre)
                part_v[i, :] = acc
                return c2

            lax.fori_loop(0, SUB, row_body, 0)
            pltpu.sync_copy(part_v, out_hbm.at[pl.ds(base + off, SUB)])
            return carry

        lax.fori_loop(0, nsub, sub_body, 0)

    return sc_kernel(ent_re, ent_im, rel_re, rel_im, h_idx, r_idx, t_idx)


def _loss_tc_kernel(part_ref, out_ref):
    x = part_ref[...]
    s = jnp.sum(x, axis=2)
    sgn = jnp.concatenate(
        [jnp.full((1, s.shape[1]), -1.0, jnp.float32),
         jnp.full((1, s.shape[1]), 1.0, jnp.float32)], axis=0)
    z = s * sgn
    sp = jnp.maximum(z, 0.0) + jnp.log1p(jnp.exp(-jnp.abs(z)))
    out_ref[...] = jnp.mean(sp, axis=(0, 1), keepdims=True).reshape(1, 1)


def kernel(ent_re, ent_im, rel_re, rel_im, positive_triples, negative_triples):
    b = positive_triples.shape[0]
    h_idx = jnp.concatenate(
        [positive_triples[:, 0], negative_triples[:, 0]]).astype(jnp.int32)
    r_idx = jnp.concatenate(
        [positive_triples[:, 1], negative_triples[:, 1]]).astype(jnp.int32)
    t_idx = jnp.concatenate(
        [positive_triples[:, 2], negative_triples[:, 2]]).astype(jnp.int32)

    part = _sc_partial_scores(ent_re, ent_im, rel_re, rel_im, h_idx, r_idx, t_idx)
    part3 = part.reshape(2, b, L)

    loss = pl.pallas_call(
        _loss_tc_kernel,
        out_shape=jax.ShapeDtypeStruct((1, 1), jnp.float32),
    )(part3)
    return loss.reshape(())
